# baseline (speedup 1.0000x reference)
import jax
import jax.numpy as jnp
from jax.experimental import pallas as pl
from jax.experimental.pallas import tpu as pltpu

_DIMS = (29, 256, 64, 32, 30)


def _mlp_kernel(x_ref, w1_hbm, b1_hbm, w2_hbm, b2_hbm, w3_hbm, b3_hbm,
                w4_hbm, b4_hbm, o_ref,
                w1s, b1s, w2s, b2s, w3s, b3s, w4s, b4s,
                s0, s1, s2, s3, s4, s5, s6, s7):
    hbm = (w1_hbm, b1_hbm, w2_hbm, b2_hbm, w3_hbm, b3_hbm, w4_hbm, b4_hbm)
    vmem = (w1s, b1s, w2s, b2s, w3s, b3s, w4s, b4s)
    sems = (s0, s1, s2, s3, s4, s5, s6, s7)

    @pl.when(pl.program_id(0) == 0)
    def _load_params():
        for src, dst, sem in zip(hbm, vmem, sems):
            pltpu.make_async_copy(src, dst, sem).start()
        for src, dst, sem in zip(hbm, vmem, sems):
            pltpu.make_async_copy(src, dst, sem).wait()

    h = x_ref[...].astype(jnp.bfloat16)

    def hidden(h, ws, bs):
        y = jnp.dot(ws[...], h, preferred_element_type=jnp.float32)
        return jnp.maximum(y.astype(jnp.bfloat16) + bs[...], 0)

    h = hidden(h, w1s, b1s)
    h = hidden(h, w2s, b2s)
    h = hidden(h, w3s, b3s)
    y = jnp.dot(w4s[...], h, preferred_element_type=jnp.float32)
    o_ref[...] = jnp.maximum(y + b4s[...], 0.0)


def kernel(x, w1, b1, w2, b2, w3, b3, w4, b4):
    batch, in_dim = x.shape
    assert in_dim == _DIMS[0]

    tbn = 16384
    assert batch % tbn == 0
    grid = (batch // tbn,)

    xt = x.T

    wts = [w.T.astype(jnp.bfloat16) for w in (w1, w2, w3, w4)]
    bts = [b.reshape(-1, 1).astype(jnp.bfloat16) for b in (b1, b2, b3)]
    bts.append(b4.reshape(-1, 1))

    x_spec = pl.BlockSpec((in_dim, tbn), lambda i: (0, i))
    out_spec = pl.BlockSpec((_DIMS[-1], tbn), lambda i: (0, i))
    any_spec = pl.BlockSpec(memory_space=pltpu.MemorySpace.HBM)

    args = [xt]
    scratch = []
    for wt, bt in zip(wts, bts):
        args.extend([wt, bt])
        scratch.extend([pltpu.VMEM(wt.shape, wt.dtype),
                        pltpu.VMEM(bt.shape, bt.dtype)])
    scratch.extend([pltpu.SemaphoreType.DMA] * 8)

    flops = 2 * batch * sum(_DIMS[i] * _DIMS[i + 1] for i in range(4))
    bytes_accessed = 4 * batch * (_DIMS[0] + _DIMS[-1])

    out_t = pl.pallas_call(
        _mlp_kernel,
        out_shape=jax.ShapeDtypeStruct((_DIMS[-1], batch), jnp.float32),
        grid=grid,
        in_specs=[x_spec] + [any_spec] * 8,
        out_specs=out_spec,
        scratch_shapes=scratch,
        compiler_params=pltpu.CompilerParams(
            dimension_semantics=("arbitrary",)),
        cost_estimate=pl.CostEstimate(flops=flops, transcendentals=0,
                                      bytes_accessed=bytes_accessed),
    )(*args)
    return out_t.T


# speedup vs baseline: 1.0164x; 1.0164x over previous
import jax
import jax.numpy as jnp
from jax.experimental import pallas as pl
from jax.experimental.pallas import tpu as pltpu

_DIMS = (29, 256, 64, 32, 30)


def _mlp_kernel(x_ref, w1_ref, b1_ref, w2_ref, b2_ref, w3_ref, b3_ref,
                w4_ref, b4_ref, o_ref):
    h = x_ref[...].astype(jnp.bfloat16)

    def hidden(h, w_ref, b_ref):
        y = jnp.dot(w_ref[...], h, preferred_element_type=jnp.float32)
        return jnp.maximum(y.astype(jnp.bfloat16) + b_ref[...], 0)

    h = hidden(h, w1_ref, b1_ref)
    h = hidden(h, w2_ref, b2_ref)
    h = hidden(h, w3_ref, b3_ref)
    y = jnp.dot(w4_ref[...], h, preferred_element_type=jnp.float32)
    o_ref[...] = jnp.maximum(y + b4_ref[...], 0.0)


def kernel(x, w1, b1, w2, b2, w3, b3, w4, b4):
    batch, in_dim = x.shape
    assert in_dim == _DIMS[0]

    tbn = 32768
    assert batch % tbn == 0
    grid = (batch // tbn,)

    xt = x.T

    wts = [w.T.astype(jnp.bfloat16) for w in (w1, w2, w3, w4)]
    bts = [b.reshape(-1, 1).astype(jnp.bfloat16) for b in (b1, b2, b3)]
    bts.append(b4.reshape(-1, 1))

    x_spec = pl.BlockSpec((in_dim, tbn), lambda i: (0, i))
    out_spec = pl.BlockSpec((_DIMS[-1], tbn), lambda i: (0, i))
    param_specs = []
    for wt, bt in zip(wts, bts):
        param_specs.append(pl.BlockSpec(wt.shape, lambda i: (0, 0)))
        param_specs.append(pl.BlockSpec(bt.shape, lambda i: (0, 0)))

    args = [xt]
    for wt, bt in zip(wts, bts):
        args.extend([wt, bt])

    flops = 2 * batch * sum(_DIMS[i] * _DIMS[i + 1] for i in range(4))
    bytes_accessed = 4 * batch * (_DIMS[0] + _DIMS[-1])

    out_t = pl.pallas_call(
        _mlp_kernel,
        out_shape=jax.ShapeDtypeStruct((_DIMS[-1], batch), jnp.float32),
        grid=grid,
        in_specs=[x_spec] + param_specs,
        out_specs=out_spec,
        compiler_params=pltpu.CompilerParams(
            dimension_semantics=("parallel",)),
        cost_estimate=pl.CostEstimate(flops=flops, transcendentals=0,
                                      bytes_accessed=bytes_accessed),
    )(*args)
    return out_t.T


# speedup vs baseline: 1.0719x; 1.0545x over previous
import jax
import jax.numpy as jnp
from jax.experimental import pallas as pl
from jax.experimental.pallas import tpu as pltpu

_DIMS = (29, 256, 64, 32, 30)


def _mlp_kernel(x_ref, w1_ref, w2_ref, w3_ref, w4_ref, bp_ref, o_ref):
    h = x_ref[...].astype(jnp.bfloat16)

    _dn = (((0,), (0,)), ((), ()))

    def hidden(h, w_ref, b_col):
        w = w_ref[...].astype(jnp.bfloat16)
        y = jax.lax.dot_general(w, h, _dn, preferred_element_type=jnp.float32)
        return jnp.maximum(y.astype(jnp.bfloat16) + b_col, 0)

    h = hidden(h, w1_ref, bp_ref[0:256, 0:1].astype(jnp.bfloat16))
    h = hidden(h, w2_ref, bp_ref[0:64, 1:2].astype(jnp.bfloat16))
    h = hidden(h, w3_ref, bp_ref[0:32, 2:3].astype(jnp.bfloat16))
    w4 = w4_ref[...].astype(jnp.bfloat16)
    y = jax.lax.dot_general(w4, h, _dn, preferred_element_type=jnp.float32)
    o_ref[...] = jnp.maximum(y + bp_ref[0:30, 3:4], 0.0)


def kernel(x, w1, b1, w2, b2, w3, b3, w4, b4):
    batch, in_dim = x.shape
    assert in_dim == _DIMS[0]

    tbn = 32768
    assert batch % tbn == 0
    grid = (batch // tbn,)

    xt = x.T

    wts = [w1, w2, w3, w4]
    bp = jnp.stack([jnp.pad(b, (0, 256 - b.shape[0])) for b in
                    (b1, b2, b3, b4)], axis=1)

    x_spec = pl.BlockSpec((in_dim, tbn), lambda i: (0, i))
    out_spec = pl.BlockSpec((_DIMS[-1], tbn), lambda i: (0, i))
    param_specs = [pl.BlockSpec(wt.shape, lambda i: (0, 0)) for wt in wts]
    param_specs.append(pl.BlockSpec(bp.shape, lambda i: (0, 0)))

    args = [xt] + wts + [bp]

    flops = 2 * batch * sum(_DIMS[i] * _DIMS[i + 1] for i in range(4))
    bytes_accessed = 4 * batch * (_DIMS[0] + _DIMS[-1])

    out_t = pl.pallas_call(
        _mlp_kernel,
        out_shape=jax.ShapeDtypeStruct((_DIMS[-1], batch), jnp.float32),
        grid=grid,
        in_specs=[x_spec] + param_specs,
        out_specs=out_spec,
        compiler_params=pltpu.CompilerParams(
            dimension_semantics=("parallel",)),
        cost_estimate=pl.CostEstimate(flops=flops, transcendentals=0,
                                      bytes_accessed=bytes_accessed),
    )(*args)
    return out_t.T
